# Initial kernel scaffold; baseline (speedup 1.0000x reference)
import jax
import jax.numpy as jnp
from jax.experimental import pallas as pl


def kernel(text_x, image_x, ta_enc_W, ta_enc_b, ia_enc_W, ia_enc_b, te_Ws, te_bs, ie_Ws, ie_bs, td_Ws, td_bs, id_Ws, id_bs, ta_dec_W, ta_dec_b, ia_dec_W, ia_dec_b, text_cb, image_cb):
    raise NotImplementedError("write your pallas kernel here")


# speedup vs baseline: 1.8141x; 1.8141x over previous
import jax
import jax.numpy as jnp
from jax.experimental import pallas as pl

_BATCH = 4096
_TILE = 512
_NCB = 4
_CBS = 256
_CBD = 64
_BETA = 0.25
_ENC = [768, 512, 256, 128, 64]


def _fwd_one_modality(x, aW, ab, eWs, ebs, dWs, dbs, zW, zb, cb_ref, cbT_ref):
    h = jnp.dot(x, aW) + ab
    n = len(eWs)
    for i in range(n):
        h = jnp.dot(h, eWs[i]) + ebs[i]
        if i < n - 1:
            h = jnp.maximum(h, 0.0)
    r = h
    xq = jnp.zeros_like(h)
    ssq = jnp.zeros((1, 1), dtype=jnp.float32)
    idxs = []
    col = jax.lax.broadcasted_iota(jnp.int32, (x.shape[0], _CBS), 1)
    for l in range(_NCB):
        cbl = cb_ref[l]
        cblT = cbT_ref[l]
        cb2 = jnp.sum(cblT * cblT, axis=0, keepdims=True)
        r2 = jnp.sum(r * r, axis=1, keepdims=True)
        d = (r2 - 2.0 * jnp.dot(r, cblT)) + cb2
        dmin = jnp.min(d, axis=1, keepdims=True)
        idx = jnp.min(jnp.where(d == dmin, col, _CBS), axis=1, keepdims=True)
        oh = (col == idx).astype(jnp.float32)
        q = jnp.dot(oh, cbl, precision=jax.lax.Precision.HIGHEST)
        diff = q - r
        ssq = ssq + jnp.sum(diff * diff, keepdims=True).reshape(1, 1)
        r = r - q
        xq = xq + q
        idxs.append(idx)
    h = xq
    for i in range(n):
        h = jnp.dot(h, dWs[i]) + dbs[i]
        if i < n - 1:
            h = jnp.maximum(h, 0.0)
    out = jnp.dot(h, zW) + zb
    e = out - x
    rec = jnp.sum(e * e, keepdims=True).reshape(1, 1)
    ind = jnp.concatenate(idxs, axis=1)
    return out, rec, ssq, ind


def _kernel_body(*refs):
    (xt_ref, xi_ref, taW, tab, iaW, iab) = refs[0:6]
    teW = refs[6:10]
    teb = refs[10:14]
    ieW = refs[14:18]
    ieb = refs[18:22]
    tdW = refs[22:26]
    tdb = refs[26:30]
    idW = refs[30:34]
    idb = refs[34:38]
    (tzW, tzb, izW, izb, tcb, tcbT, icb, icbT) = refs[38:46]
    (out_t_ref, out_i_ref, total_ref, ind_t_ref, ind_i_ref) = refs[46:51]

    out_t, rec_t, ssq_t, ind_t = _fwd_one_modality(
        xt_ref[...], taW[...], tab[...],
        [w[...] for w in teW], [b[...] for b in teb],
        [w[...] for w in tdW], [b[...] for b in tdb],
        tzW[...], tzb[...], tcb, tcbT)
    out_i, rec_i, ssq_i, ind_i = _fwd_one_modality(
        xi_ref[...], iaW[...], iab[...],
        [w[...] for w in ieW], [b[...] for b in ieb],
        [w[...] for w in idW], [b[...] for b in idb],
        izW[...], izb[...], icb, icbT)

    out_t_ref[...] = out_t
    out_i_ref[...] = out_i
    ind_t_ref[...] = ind_t
    ind_i_ref[...] = ind_i

    c_rec = 1.0 / (_BATCH * _ENC[0])
    c_q = (1.0 + _BETA) / (_NCB * _BATCH * _CBD)
    partial = (rec_t + rec_i) * c_rec + (ssq_t + ssq_i) * c_q

    i = pl.program_id(0)

    @pl.when(i == 0)
    def _():
        total_ref[...] = partial

    @pl.when(i > 0)
    def _():
        total_ref[...] = total_ref[...] + partial


def _full_spec(shape):
    nd = len(shape)
    return pl.BlockSpec(shape, lambda i, _nd=nd: (0,) * _nd)


def _build_call():
    nt = _BATCH // _TILE
    row_spec = pl.BlockSpec((_TILE, _ENC[0]), lambda i: (i, 0))
    ind_spec = pl.BlockSpec((_TILE, _NCB), lambda i: (i, 0))

    in_specs = [row_spec, row_spec]
    in_specs += [_full_spec((768, 768)), _full_spec((1, 768)),
                 _full_spec((768, 768)), _full_spec((1, 768))]
    enc_w_shapes = [(_ENC[i], _ENC[i + 1]) for i in range(4)]
    dec_w_shapes = [(_ENC[4 - i], _ENC[3 - i]) for i in range(4)]
    enc_b_shapes = [(1, _ENC[i + 1]) for i in range(4)]
    dec_b_shapes = [(1, _ENC[3 - i]) for i in range(4)]
    for shapes in (enc_w_shapes, enc_b_shapes, enc_w_shapes, enc_b_shapes,
                   dec_w_shapes, dec_b_shapes, dec_w_shapes, dec_b_shapes):
        in_specs += [_full_spec(s) for s in shapes]
    in_specs += [_full_spec((768, 768)), _full_spec((1, 768)),
                 _full_spec((768, 768)), _full_spec((1, 768))]
    in_specs += [_full_spec((_NCB, _CBS, _CBD)), _full_spec((_NCB, _CBD, _CBS)),
                 _full_spec((_NCB, _CBS, _CBD)), _full_spec((_NCB, _CBD, _CBS))]

    out_specs = [row_spec, row_spec,
                 pl.BlockSpec((1, 1), lambda i: (0, 0)),
                 ind_spec, ind_spec]
    out_shape = [
        jax.ShapeDtypeStruct((_BATCH, 768), jnp.float32),
        jax.ShapeDtypeStruct((_BATCH, 768), jnp.float32),
        jax.ShapeDtypeStruct((1, 1), jnp.float32),
        jax.ShapeDtypeStruct((_BATCH, _NCB), jnp.int32),
        jax.ShapeDtypeStruct((_BATCH, _NCB), jnp.int32),
    ]
    return pl.pallas_call(
        _kernel_body,
        grid=(nt,),
        in_specs=in_specs,
        out_specs=out_specs,
        out_shape=out_shape,
    )


def kernel(text_x, image_x, ta_enc_W, ta_enc_b, ia_enc_W, ia_enc_b,
           te_Ws, te_bs, ie_Ws, ie_bs, td_Ws, td_bs, id_Ws, id_bs,
           ta_dec_W, ta_dec_b, ia_dec_W, ia_dec_b, text_cb, image_cb):
    r2 = lambda b: b.reshape(1, -1)
    args = [text_x, image_x, ta_enc_W, r2(ta_enc_b), ia_enc_W, r2(ia_enc_b)]
    args += list(te_Ws) + [r2(b) for b in te_bs]
    args += list(ie_Ws) + [r2(b) for b in ie_bs]
    args += list(td_Ws) + [r2(b) for b in td_bs]
    args += list(id_Ws) + [r2(b) for b in id_bs]
    args += [ta_dec_W, r2(ta_dec_b), ia_dec_W, r2(ia_dec_b)]
    args += [text_cb, jnp.swapaxes(text_cb, 1, 2),
             image_cb, jnp.swapaxes(image_cb, 1, 2)]
    out_t, out_i, total, ind_t, ind_i = _build_call()(*args)
    return (out_t, out_i, total.reshape(()), ind_t, ind_i)


# speedup vs baseline: 1.9217x; 1.0593x over previous
import jax
import jax.numpy as jnp
from jax.experimental import pallas as pl

_BATCH = 4096
_TILE = 512
_NCB = 4
_CBS = 256
_CBD = 64
_BETA = 0.25
_ENC = [768, 512, 256, 128, 64]

_BF = jnp.bfloat16
_F32 = jnp.float32


def _dot(a, b):
    return jax.lax.dot_general(a.astype(_BF), b, (((1,), (0,)), ((), ())),
                               preferred_element_type=_F32)


def _fwd_one_modality(x, aW, ab, eWs, ebs, dWs, dbs, zW, zb, cb3_ref, cbT_ref):
    h = _dot(x, aW) + ab
    n = len(eWs)
    for i in range(n):
        h = _dot(h, eWs[i]) + ebs[i]
        if i < n - 1:
            h = jnp.maximum(h, 0.0)
    r = h
    xq = jnp.zeros_like(h)
    ssq = jnp.zeros((1, 1), dtype=_F32)
    idxs = []
    col = jax.lax.broadcasted_iota(jnp.int32, (x.shape[0], _CBS), 1)
    for l in range(_NCB):
        cblT = cbT_ref[l]
        cbl3 = cb3_ref[l]
        cb2 = jnp.sum(cblT * cblT, axis=0, keepdims=True)
        r2 = jnp.sum(r * r, axis=1, keepdims=True)
        d = (r2 - 2.0 * _dot(r, cblT.astype(_BF))) + cb2
        dmin = jnp.min(d, axis=1, keepdims=True)
        idx = jnp.min(jnp.where(d == dmin, col, _CBS), axis=1, keepdims=True)
        oh = (col == idx).astype(_BF)
        q3 = jax.lax.dot_general(oh, cbl3, (((1,), (0,)), ((), ())),
                                 preferred_element_type=_F32)
        q = (q3[:, 0:_CBD] + q3[:, _CBD:2 * _CBD]) + q3[:, 2 * _CBD:3 * _CBD]
        diff = q - r
        ssq = ssq + jnp.sum(diff * diff, keepdims=True).reshape(1, 1)
        r = r - q
        xq = xq + q
        idxs.append(idx)
    h = xq
    for i in range(n):
        h = _dot(h, dWs[i]) + dbs[i]
        if i < n - 1:
            h = jnp.maximum(h, 0.0)
    out = _dot(h, zW) + zb
    e = out - x
    rec = jnp.sum(e * e, keepdims=True).reshape(1, 1)
    ind = jnp.concatenate(idxs, axis=1)
    return out, rec, ssq, ind


def _kernel_body(*refs):
    (xt_ref, xi_ref, taW, tab, iaW, iab) = refs[0:6]
    teW = refs[6:10]
    teb = refs[10:14]
    ieW = refs[14:18]
    ieb = refs[18:22]
    tdW = refs[22:26]
    tdb = refs[26:30]
    idW = refs[30:34]
    idb = refs[34:38]
    (tzW, tzb, izW, izb, tcb3, tcbT, icb3, icbT) = refs[38:46]
    (out_t_ref, out_i_ref, total_ref, ind_t_ref, ind_i_ref) = refs[46:51]

    out_t, rec_t, ssq_t, ind_t = _fwd_one_modality(
        xt_ref[...], taW[...], tab[...],
        [w[...] for w in teW], [b[...] for b in teb],
        [w[...] for w in tdW], [b[...] for b in tdb],
        tzW[...], tzb[...], tcb3, tcbT)
    out_i, rec_i, ssq_i, ind_i = _fwd_one_modality(
        xi_ref[...], iaW[...], iab[...],
        [w[...] for w in ieW], [b[...] for b in ieb],
        [w[...] for w in idW], [b[...] for b in idb],
        izW[...], izb[...], icb3, icbT)

    out_t_ref[...] = out_t
    out_i_ref[...] = out_i
    ind_t_ref[...] = ind_t
    ind_i_ref[...] = ind_i

    c_rec = 1.0 / (_BATCH * _ENC[0])
    c_q = (1.0 + _BETA) / (_NCB * _BATCH * _CBD)
    partial = (rec_t + rec_i) * c_rec + (ssq_t + ssq_i) * c_q

    i = pl.program_id(0)

    @pl.when(i == 0)
    def _():
        total_ref[...] = partial

    @pl.when(i > 0)
    def _():
        total_ref[...] = total_ref[...] + partial


def _full_spec(shape):
    nd = len(shape)
    return pl.BlockSpec(shape, lambda i, _nd=nd: (0,) * _nd)


def _build_call():
    nt = _BATCH // _TILE
    row_spec = pl.BlockSpec((_TILE, _ENC[0]), lambda i: (i, 0))
    ind_spec = pl.BlockSpec((_TILE, _NCB), lambda i: (i, 0))

    in_specs = [row_spec, row_spec]
    in_specs += [_full_spec((768, 768)), _full_spec((1, 768)),
                 _full_spec((768, 768)), _full_spec((1, 768))]
    enc_w_shapes = [(_ENC[i], _ENC[i + 1]) for i in range(4)]
    dec_w_shapes = [(_ENC[4 - i], _ENC[3 - i]) for i in range(4)]
    enc_b_shapes = [(1, _ENC[i + 1]) for i in range(4)]
    dec_b_shapes = [(1, _ENC[3 - i]) for i in range(4)]
    for shapes in (enc_w_shapes, enc_b_shapes, enc_w_shapes, enc_b_shapes,
                   dec_w_shapes, dec_b_shapes, dec_w_shapes, dec_b_shapes):
        in_specs += [_full_spec(s) for s in shapes]
    in_specs += [_full_spec((768, 768)), _full_spec((1, 768)),
                 _full_spec((768, 768)), _full_spec((1, 768))]
    in_specs += [_full_spec((_NCB, _CBS, 3 * _CBD)), _full_spec((_NCB, _CBD, _CBS)),
                 _full_spec((_NCB, _CBS, 3 * _CBD)), _full_spec((_NCB, _CBD, _CBS))]

    out_specs = [row_spec, row_spec,
                 pl.BlockSpec((1, 1), lambda i: (0, 0)),
                 ind_spec, ind_spec]
    out_shape = [
        jax.ShapeDtypeStruct((_BATCH, 768), _F32),
        jax.ShapeDtypeStruct((_BATCH, 768), _F32),
        jax.ShapeDtypeStruct((1, 1), _F32),
        jax.ShapeDtypeStruct((_BATCH, _NCB), jnp.int32),
        jax.ShapeDtypeStruct((_BATCH, _NCB), jnp.int32),
    ]
    return pl.pallas_call(
        _kernel_body,
        grid=(nt,),
        in_specs=in_specs,
        out_specs=out_specs,
        out_shape=out_shape,
    )


def _cb_components(cb):
    hi = cb.astype(_BF)
    rem = cb - hi.astype(_F32)
    mid = rem.astype(_BF)
    lo = (rem - mid.astype(_F32)).astype(_BF)
    return jnp.concatenate([hi, mid, lo], axis=2)


def kernel(text_x, image_x, ta_enc_W, ta_enc_b, ia_enc_W, ia_enc_b,
           te_Ws, te_bs, ie_Ws, ie_bs, td_Ws, td_bs, id_Ws, id_bs,
           ta_dec_W, ta_dec_b, ia_dec_W, ia_dec_b, text_cb, image_cb):
    r2 = lambda b: b.reshape(1, -1)
    w = lambda W: W.astype(_BF)
    args = [text_x, image_x, w(ta_enc_W), r2(ta_enc_b), w(ia_enc_W), r2(ia_enc_b)]
    args += [w(W) for W in te_Ws] + [r2(b) for b in te_bs]
    args += [w(W) for W in ie_Ws] + [r2(b) for b in ie_bs]
    args += [w(W) for W in td_Ws] + [r2(b) for b in td_bs]
    args += [w(W) for W in id_Ws] + [r2(b) for b in id_bs]
    args += [w(ta_dec_W), r2(ta_dec_b), w(ia_dec_W), r2(ia_dec_b)]
    args += [_cb_components(text_cb), jnp.swapaxes(text_cb, 1, 2),
             _cb_components(image_cb), jnp.swapaxes(image_cb, 1, 2)]
    out_t, out_i, total, ind_t, ind_i = _build_call()(*args)
    return (out_t, out_i, total.reshape(()), ind_t, ind_i)
